# Initial kernel scaffold; baseline (speedup 1.0000x reference)
import jax
import jax.numpy as jnp
from jax.experimental import pallas as pl


def kernel(x, edge_index, W, b, gamma, beta):
    raise NotImplementedError("write your pallas kernel here")


# speedup vs baseline: 15.9869x; 15.9869x over previous
import functools

import jax
import jax.numpy as jnp
from jax import lax
from jax.experimental import pallas as pl
from jax.experimental.pallas import tpu as pltpu
from jax.experimental.pallas import tpu_sc as plsc

N = 10000
E = 320000
D = 128

NC = 2
NS = 16
NW = NC * NS
EPW = E // NW
CH = 80
NCHUNK = EPW // CH
N_PAD = 10240
ROWS_PER_SUB = N_PAD // NS

_mesh = lambda: plsc.VectorSubcoreMesh(core_axis_name="c", subcore_axis_name="s")


@functools.partial(
    pl.kernel,
    out_type=jax.ShapeDtypeStruct((NC, N), jnp.float32),
    mesh=_mesh(),
    scratch_types=[
        pltpu.VMEM((CH,), jnp.int32),
        pltpu.VMEM((CH,), jnp.float32),
        pltpu.VMEM_SHARED((N,), jnp.float32),
    ],
)
def _deg_kernel(dst_hbm, zeros_hbm, out_hbm, idx_v, ones_v, acc_sh):
  c = lax.axis_index("c")
  s = lax.axis_index("s")
  wid = s * NC + c

  for j in range(CH // 16):
    ones_v[pl.ds(j * 16, 16)] = jnp.full((16,), 1.0, jnp.float32)

  @pl.when(s == 0)
  def _():
    pltpu.sync_copy(zeros_hbm, acc_sh)
  plsc.subcore_barrier()

  base = wid * EPW

  def body(i, carry):
    off = base + i * CH
    pltpu.sync_copy(dst_hbm.at[pl.ds(off, CH)], idx_v)
    pltpu.sync_copy(ones_v, acc_sh.at[idx_v], add=True)
    return carry

  lax.fori_loop(0, NCHUNK, body, 0)
  plsc.subcore_barrier()

  @pl.when(s == 0)
  def _():
    pltpu.sync_copy(acc_sh, out_hbm.at[c])


@functools.partial(
    pl.kernel,
    out_type=jax.ShapeDtypeStruct((NC, N_PAD, D), jnp.float32),
    mesh=_mesh(),
    scratch_types=[
        pltpu.VMEM((CH,), jnp.int32),
        pltpu.VMEM((CH,), jnp.int32),
        pltpu.VMEM((CH, D), jnp.float32),
        pltpu.VMEM_SHARED((N_PAD, D), jnp.float32),
        pltpu.SemaphoreType.DMA,
    ],
)
def _agg_kernel(src_hbm, dst_hbm, xp_hbm, zeros_hbm, out_hbm,
                sidx_v, didx_v, rows_v, acc_sh, sem):
  c = lax.axis_index("c")
  s = lax.axis_index("s")
  wid = s * NC + c

  r0 = s * ROWS_PER_SUB
  pltpu.sync_copy(zeros_hbm.at[pl.ds(r0, ROWS_PER_SUB)],
                  acc_sh.at[pl.ds(r0, ROWS_PER_SUB)])
  plsc.subcore_barrier()

  base = wid * EPW

  def body(i, carry):
    off = base + i * CH
    pltpu.sync_copy(src_hbm.at[pl.ds(off, CH)], sidx_v)
    pltpu.sync_copy(dst_hbm.at[pl.ds(off, CH)], didx_v)
    pltpu.async_copy(xp_hbm.at[sidx_v], rows_v, sem).wait()
    pltpu.sync_copy(rows_v, acc_sh.at[didx_v], add=True)
    return carry

  lax.fori_loop(0, NCHUNK, body, 0)
  plsc.subcore_barrier()

  pltpu.sync_copy(acc_sh.at[pl.ds(r0, ROWS_PER_SUB)],
                  out_hbm.at[c, pl.ds(r0, ROWS_PER_SUB)])


def _scale_body(x_ref, dp_ref, o_ref):
  deg = 1.0 + dp_ref[:, 0:1] + dp_ref[:, 1:2]
  o_ref[...] = x_ref[...] * lax.rsqrt(deg)


def _scale_tc(x, dp):
  R = 2000
  grid = (N // R,)
  return pl.pallas_call(
      _scale_body,
      grid=grid,
      in_specs=[
          pl.BlockSpec((R, D), lambda i: (i, 0)),
          pl.BlockSpec((R, 2), lambda i: (i, 0)),
      ],
      out_specs=pl.BlockSpec((R, D), lambda i: (i, 0)),
      out_shape=jax.ShapeDtypeStruct((N, D), jnp.float32),
  )(x, dp)


def _finish_body(x_ref, p0_ref, p1_ref, dp_ref, w_ref, b_ref, g_ref, be_ref,
                 o_ref):
  deg = 1.0 + dp_ref[:, 0:1] + dp_ref[:, 1:2]
  dinv = lax.rsqrt(deg)
  x = x_ref[...]
  sagg = (p0_ref[...] + p1_ref[...] + x * dinv) * dinv
  y = jnp.dot(sagg, w_ref[...], preferred_element_type=jnp.float32)
  y = y + b_ref[...] + x
  mean = jnp.mean(y, axis=-1, keepdims=True)
  yc = y - mean
  var = jnp.mean(yc * yc, axis=-1, keepdims=True)
  yn = yc * lax.rsqrt(var + 1e-5) * g_ref[...] + be_ref[...]
  o_ref[...] = jnp.maximum(yn, 0.0)


def _finish_tc(x, p0, p1, dp, W, b, gamma, beta):
  R = 2000
  grid = (N // R,)
  row = lambda i: (i, 0)
  full = lambda i: (0, 0)
  return pl.pallas_call(
      _finish_body,
      grid=grid,
      in_specs=[
          pl.BlockSpec((R, D), row),
          pl.BlockSpec((R, D), row),
          pl.BlockSpec((R, D), row),
          pl.BlockSpec((R, 2), row),
          pl.BlockSpec((D, D), full),
          pl.BlockSpec((1, D), full),
          pl.BlockSpec((1, D), full),
          pl.BlockSpec((1, D), full),
      ],
      out_specs=pl.BlockSpec((R, D), row),
      out_shape=jax.ShapeDtypeStruct((N, D), jnp.float32),
  )(x, p0, p1, dp, W, b, gamma, beta)


def kernel(x, edge_index, W, b, gamma, beta):
  ei = edge_index.astype(jnp.int32)
  src = ei[0]
  dst = ei[1]
  zeros_nd = jnp.zeros((N_PAD, D), jnp.float32)
  zeros_n = jnp.zeros((N,), jnp.float32)

  degp = _deg_kernel(dst, zeros_n)
  dp = degp.T
  xp = _scale_tc(x, dp)
  accp = _agg_kernel(src, dst, xp, zeros_nd)
  return _finish_tc(x, accp[0, :N], accp[1, :N], dp, W,
                    b.reshape(1, D), gamma.reshape(1, D), beta.reshape(1, D))


# speedup vs baseline: 16.1157x; 1.0081x over previous
import functools

import jax
import jax.numpy as jnp
from jax import lax
from jax.experimental import pallas as pl
from jax.experimental.pallas import tpu as pltpu
from jax.experimental.pallas import tpu_sc as plsc

N = 10000
E = 320000
D = 128
DH = D // 2

NC = 2
NS = 16
NW = NC * NS
CH = 128
NCHUNKS = 2560
E_PAD = NCHUNKS * CH
CHT = NCHUNKS // NS
CHW = NCHUNKS // NW
N_PAD = 10240
ROWS_PER_SUB = N_PAD // NS

_mesh = lambda: plsc.VectorSubcoreMesh(core_axis_name="c", subcore_axis_name="s")


@functools.partial(
    pl.kernel,
    out_type=jax.ShapeDtypeStruct((NC, N_PAD), jnp.float32),
    mesh=_mesh(),
    scratch_types=[
        pltpu.VMEM((CHW, CH), jnp.int32),
        pltpu.VMEM((CH,), jnp.float32),
        pltpu.VMEM_SHARED((N_PAD,), jnp.float32),
        pltpu.SemaphoreType.DMA,
    ],
)
def _deg_kernel(dst_hbm, zeros_hbm, out_hbm, didx_v, ones_v, acc_sh, sem):
  c = lax.axis_index("c")
  s = lax.axis_index("s")
  wid = s * NC + c

  for j in range(CH // 16):
    ones_v[pl.ds(j * 16, 16)] = jnp.full((16,), 1.0, jnp.float32)

  @pl.when(s == 0)
  def _():
    pltpu.sync_copy(zeros_hbm, acc_sh)
  pltpu.sync_copy(dst_hbm.at[pl.ds(wid * CHW, CHW)], didx_v)
  plsc.subcore_barrier()

  G = 5

  def body(g, carry):
    j0 = g * G
    for j in range(G):
      pltpu.async_copy(ones_v, acc_sh.at[didx_v.at[j0 + j]], sem, add=True)
    for j in range(G):
      pltpu.make_async_copy(ones_v, acc_sh.at[didx_v.at[j0 + j]], sem).wait()
    return carry

  lax.fori_loop(0, CHW // G, body, 0)
  plsc.subcore_barrier()

  @pl.when(s == 0)
  def _():
    pltpu.sync_copy(acc_sh, out_hbm.at[c])


@functools.partial(
    pl.kernel,
    out_type=jax.ShapeDtypeStruct((NC, N_PAD, DH), jnp.float32),
    mesh=_mesh(),
    compiler_params=pltpu.CompilerParams(use_tc_tiling_on_sc=False),
    scratch_types=[
        pltpu.VMEM((CHT, CH), jnp.int32),
        pltpu.VMEM((CHT, CH), jnp.int32),
        pltpu.VMEM((CH, DH), jnp.float32),
        pltpu.VMEM((CH, DH), jnp.float32),
        pltpu.VMEM_SHARED((N_PAD, DH), jnp.float32),
        pltpu.SemaphoreType.DMA,
        pltpu.SemaphoreType.DMA,
    ],
)
def _agg_kernel(src_hbm, dst_hbm, xp0_hbm, xp1_hbm, zeros_hbm, out_hbm,
                sidx_v, didx_v, rows0, rows1, acc_sh, sem0, sem1):
  c = lax.axis_index("c")
  s = lax.axis_index("s")

  r0 = s * ROWS_PER_SUB
  pltpu.sync_copy(zeros_hbm.at[pl.ds(r0, ROWS_PER_SUB)],
                  acc_sh.at[pl.ds(r0, ROWS_PER_SUB)])
  pltpu.sync_copy(src_hbm.at[pl.ds(s * CHT, CHT)], sidx_v)
  pltpu.sync_copy(dst_hbm.at[pl.ds(s * CHT, CHT)], didx_v)
  plsc.subcore_barrier()

  def run(xp_hbm):
    def gstart(j, rows):
      pltpu.async_copy(xp_hbm.at[sidx_v.at[j]], rows, sem0)

    def gwait(j, rows):
      pltpu.make_async_copy(xp_hbm.at[sidx_v.at[j]], rows, sem0).wait()

    def sstart(j, rows):
      pltpu.async_copy(rows, acc_sh.at[didx_v.at[j]], sem1, add=True)

    def swait(j, rows):
      pltpu.make_async_copy(rows, acc_sh.at[didx_v.at[j]], sem1).wait()

    gstart(0, rows0)

    def body(jp, carry):
      a = jp * 2
      gwait(a, rows0)
      sstart(a, rows0)
      gstart(a + 1, rows1)
      gwait(a + 1, rows1)
      swait(a, rows0)
      sstart(a + 1, rows1)
      gstart(a + 2, rows0)
      swait(a + 1, rows1)
      return carry

    lax.fori_loop(0, CHT // 2 - 1, body, 0)
    a = CHT - 2
    gwait(a, rows0)
    sstart(a, rows0)
    gstart(a + 1, rows1)
    gwait(a + 1, rows1)
    swait(a, rows0)
    sstart(a + 1, rows1)
    swait(a + 1, rows1)

  @pl.when(c == 0)
  def _():
    run(xp0_hbm)

  @pl.when(c == 1)
  def _():
    run(xp1_hbm)

  plsc.subcore_barrier()
  pltpu.sync_copy(acc_sh.at[pl.ds(r0, ROWS_PER_SUB)],
                  out_hbm.at[c, pl.ds(r0, ROWS_PER_SUB)])


def _scale_body(x_ref, dp_ref, o0_ref, o1_ref):
  deg = 1.0 + dp_ref[:, 0:1] + dp_ref[:, 1:2]
  xp = x_ref[...] * lax.rsqrt(deg)
  o0_ref[...] = xp[:, :DH]
  o1_ref[...] = xp[:, DH:]


def _scale_tc(x, dp):
  R = 2000
  grid = (N // R,)
  return pl.pallas_call(
      _scale_body,
      grid=grid,
      in_specs=[
          pl.BlockSpec((R, D), lambda i: (i, 0)),
          pl.BlockSpec((R, 2), lambda i: (i, 0)),
      ],
      out_specs=[
          pl.BlockSpec((R, DH), lambda i: (i, 0)),
          pl.BlockSpec((R, DH), lambda i: (i, 0)),
      ],
      out_shape=[
          jax.ShapeDtypeStruct((N, DH), jnp.float32),
          jax.ShapeDtypeStruct((N, DH), jnp.float32),
      ],
  )(x, dp)


def _finish_body(x_ref, p0_ref, p1_ref, dp_ref, w_ref, b_ref, g_ref, be_ref,
                 o_ref):
  deg = 1.0 + dp_ref[:, 0:1] + dp_ref[:, 1:2]
  dinv = lax.rsqrt(deg)
  x = x_ref[...]
  p = jnp.concatenate([p0_ref[...], p1_ref[...]], axis=-1)
  sagg = (p + x * dinv) * dinv
  y = jnp.dot(sagg, w_ref[...], preferred_element_type=jnp.float32)
  y = y + b_ref[...] + x
  mean = jnp.mean(y, axis=-1, keepdims=True)
  yc = y - mean
  var = jnp.mean(yc * yc, axis=-1, keepdims=True)
  yn = yc * lax.rsqrt(var + 1e-5) * g_ref[...] + be_ref[...]
  o_ref[...] = jnp.maximum(yn, 0.0)


def _finish_tc(x, p0, p1, dp, W, b, gamma, beta):
  R = 2000
  grid = (N // R,)
  row = lambda i: (i, 0)
  full = lambda i: (0, 0)
  return pl.pallas_call(
      _finish_body,
      grid=grid,
      in_specs=[
          pl.BlockSpec((R, D), row),
          pl.BlockSpec((R, DH), row),
          pl.BlockSpec((R, DH), row),
          pl.BlockSpec((R, 2), row),
          pl.BlockSpec((D, D), full),
          pl.BlockSpec((1, D), full),
          pl.BlockSpec((1, D), full),
          pl.BlockSpec((1, D), full),
      ],
      out_specs=pl.BlockSpec((R, D), row),
      out_shape=jax.ShapeDtypeStruct((N, D), jnp.float32),
  )(x, p0, p1, dp, W, b, gamma, beta)


def kernel(x, edge_index, W, b, gamma, beta):
  ei = edge_index.astype(jnp.int32)
  pad = E_PAD - E
  src = jnp.concatenate([ei[0], jnp.zeros((pad,), jnp.int32)])
  dst = jnp.concatenate([ei[1], jnp.full((pad,), N, jnp.int32)])
  src = src.reshape(NCHUNKS, CH)
  dst = dst.reshape(NCHUNKS, CH)
  zeros_nd = jnp.zeros((N_PAD, DH), jnp.float32)
  zeros_n = jnp.zeros((N_PAD,), jnp.float32)

  degp = _deg_kernel(dst, zeros_n)
  dp = degp[:, :N].T
  xp0, xp1 = _scale_tc(x, dp)
  accp = _agg_kernel(src, dst, xp0, xp1, zeros_nd)
  return _finish_tc(x, accp[0, :N], accp[1, :N], dp, W,
                    b.reshape(1, D), gamma.reshape(1, D), beta.reshape(1, D))


# speedup vs baseline: 16.8698x; 1.0468x over previous
import functools

import jax
import jax.numpy as jnp
from jax import lax
from jax.experimental import pallas as pl
from jax.experimental.pallas import tpu as pltpu
from jax.experimental.pallas import tpu_sc as plsc

N = 10000
E = 320000
D = 128

NC = 2
NS = 16
NW = NC * NS
CH = 80
NCHUNK = 125
CHA = 128
NCHUNKA = 79
E_PAD = NW * NCHUNKA * CHA
N_PAD = 10240
ROWS_PER_SUB = N_PAD // NS

_mesh = lambda: plsc.VectorSubcoreMesh(core_axis_name="c", subcore_axis_name="s")


@functools.partial(
    pl.kernel,
    out_type=jax.ShapeDtypeStruct((NC, N_PAD), jnp.float32),
    mesh=_mesh(),
    scratch_types=[
        pltpu.VMEM((NCHUNK, CH), jnp.int32),
        pltpu.VMEM((CH,), jnp.float32),
        pltpu.VMEM_SHARED((N_PAD,), jnp.float32),
        pltpu.SemaphoreType.DMA,
    ],
)
def _deg_kernel(dst_hbm, zeros_hbm, out_hbm, didx_v, ones_v, acc_sh, sem):
  c = lax.axis_index("c")
  s = lax.axis_index("s")
  wid = s * NC + c

  for j in range(CH // 16):
    ones_v[pl.ds(j * 16, 16)] = jnp.full((16,), 1.0, jnp.float32)

  @pl.when(s == 0)
  def _():
    pltpu.sync_copy(zeros_hbm, acc_sh)
  pltpu.sync_copy(dst_hbm.at[wid], didx_v)
  plsc.subcore_barrier()

  G = 5

  def body(g, carry):
    j0 = g * G
    for j in range(G):
      pltpu.async_copy(ones_v, acc_sh.at[didx_v.at[j0 + j]], sem, add=True)
    for j in range(G):
      pltpu.make_async_copy(ones_v, acc_sh.at[didx_v.at[j0 + j]], sem).wait()
    return carry

  lax.fori_loop(0, NCHUNK // G, body, 0)
  plsc.subcore_barrier()

  @pl.when(s == 0)
  def _():
    pltpu.sync_copy(acc_sh, out_hbm.at[c])


@functools.partial(
    pl.kernel,
    out_type=jax.ShapeDtypeStruct((NC, N_PAD, D), jnp.float32),
    mesh=_mesh(),
    scratch_types=[
        pltpu.VMEM((NCHUNKA, CHA), jnp.int32),
        pltpu.VMEM((NCHUNKA, CHA), jnp.int32),
        pltpu.VMEM((CHA, D), jnp.float32),
        pltpu.VMEM_SHARED((N_PAD, D), jnp.float32),
        pltpu.SemaphoreType.DMA,
    ],
)
def _agg_kernel(src_hbm, dst_hbm, xp_hbm, zeros_hbm, out_hbm,
                sidx_v, didx_v, rows_v, acc_sh, sem0):
  c = lax.axis_index("c")
  s = lax.axis_index("s")
  wid = s * NC + c

  r0 = s * ROWS_PER_SUB
  pltpu.sync_copy(zeros_hbm.at[pl.ds(r0, ROWS_PER_SUB)],
                  acc_sh.at[pl.ds(r0, ROWS_PER_SUB)])
  pltpu.sync_copy(src_hbm.at[wid], sidx_v)
  pltpu.sync_copy(dst_hbm.at[wid], didx_v)
  plsc.subcore_barrier()

  def body(j, carry):
    pltpu.async_copy(xp_hbm.at[sidx_v.at[j]], rows_v, sem0).wait()
    pltpu.sync_copy(rows_v, acc_sh.at[didx_v.at[j]], add=True)
    return carry

  lax.fori_loop(0, NCHUNKA, body, 0)
  plsc.subcore_barrier()

  pltpu.sync_copy(acc_sh.at[pl.ds(r0, ROWS_PER_SUB)],
                  out_hbm.at[c, pl.ds(r0, ROWS_PER_SUB)])


def _scale_body(x_ref, dp_ref, o_ref):
  deg = 1.0 + dp_ref[:, 0:1] + dp_ref[:, 1:2]
  o_ref[...] = x_ref[...] * lax.rsqrt(deg)


def _scale_tc(x, dp):
  R = 2000
  grid = (N // R,)
  return pl.pallas_call(
      _scale_body,
      grid=grid,
      in_specs=[
          pl.BlockSpec((R, D), lambda i: (i, 0)),
          pl.BlockSpec((R, 2), lambda i: (i, 0)),
      ],
      out_specs=pl.BlockSpec((R, D), lambda i: (i, 0)),
      out_shape=jax.ShapeDtypeStruct((N, D), jnp.float32),
  )(x, dp)


def _finish_body(x_ref, p_ref, dp_ref, w_ref, b_ref, g_ref, be_ref, o_ref):
  deg = 1.0 + dp_ref[:, 0:1] + dp_ref[:, 1:2]
  dinv = lax.rsqrt(deg)
  x = x_ref[...]
  sagg = (p_ref[0] + p_ref[1] + x * dinv) * dinv
  y = jnp.dot(sagg, w_ref[...], preferred_element_type=jnp.float32)
  y = y + b_ref[...] + x
  mean = jnp.mean(y, axis=-1, keepdims=True)
  yc = y - mean
  var = jnp.mean(yc * yc, axis=-1, keepdims=True)
  yn = yc * lax.rsqrt(var + 1e-5) * g_ref[...] + be_ref[...]
  o_ref[...] = jnp.maximum(yn, 0.0)


def _finish_tc(x, accp, dp, W, b, gamma, beta):
  R = 2000
  grid = (N // R,)
  row = lambda i: (i, 0)
  full = lambda i: (0, 0)
  return pl.pallas_call(
      _finish_body,
      grid=grid,
      in_specs=[
          pl.BlockSpec((R, D), row),
          pl.BlockSpec((NC, R, D), lambda i: (0, i, 0)),
          pl.BlockSpec((R, 2), row),
          pl.BlockSpec((D, D), full),
          pl.BlockSpec((1, D), full),
          pl.BlockSpec((1, D), full),
          pl.BlockSpec((1, D), full),
      ],
      out_specs=pl.BlockSpec((R, D), row),
      out_shape=jax.ShapeDtypeStruct((N, D), jnp.float32),
  )(x, accp, dp, W, b, gamma, beta)


def kernel(x, edge_index, W, b, gamma, beta):
  ei = edge_index.astype(jnp.int32)
  dstd = ei[1].reshape(NW, NCHUNK, CH)
  pad = E_PAD - E
  srcp = jnp.concatenate([ei[0], jnp.zeros((pad,), jnp.int32)])
  dstp = jnp.concatenate([ei[1], jnp.full((pad,), N, jnp.int32)])
  srcp = srcp.reshape(NW, NCHUNKA, CHA)
  dstp = dstp.reshape(NW, NCHUNKA, CHA)
  zeros_nd = jnp.zeros((N_PAD, D), jnp.float32)
  zeros_n = jnp.zeros((N_PAD,), jnp.float32)

  degp = _deg_kernel(dstd, zeros_n)
  dp = degp[:, :N].T
  xp = _scale_tc(x, dp)
  accp = _agg_kernel(srcp, dstp, xp, zeros_nd)
  return _finish_tc(x, accp, dp, W,
                    b.reshape(1, D), gamma.reshape(1, D), beta.reshape(1, D))


# speedup vs baseline: 17.1635x; 1.0174x over previous
import functools

import jax
import jax.numpy as jnp
from jax import lax
from jax.experimental import pallas as pl
from jax.experimental.pallas import tpu as pltpu
from jax.experimental.pallas import tpu_sc as plsc

N = 10000
E = 320000
D = 128

NC = 2
NS = 16
NW = NC * NS
CH = 80
NCHUNK = 125
CHA = 128
NCHUNKA = 79
E_PAD = NW * NCHUNKA * CHA
N_PAD = 10240
ROWS_PER_SUB = N_PAD // NS

_mesh = lambda: plsc.VectorSubcoreMesh(core_axis_name="c", subcore_axis_name="s")


@functools.partial(
    pl.kernel,
    out_type=jax.ShapeDtypeStruct((NC, N_PAD), jnp.float32),
    mesh=_mesh(),
    scratch_types=[
        pltpu.VMEM((NCHUNK, CH), jnp.int32),
        pltpu.VMEM((CH,), jnp.float32),
        pltpu.VMEM_SHARED((N_PAD,), jnp.float32),
        pltpu.SemaphoreType.DMA,
    ],
)
def _deg_kernel(dst_hbm, zeros_hbm, out_hbm, didx_v, ones_v, acc_sh, sem):
  c = lax.axis_index("c")
  s = lax.axis_index("s")
  wid = s * NC + c

  for j in range(CH // 16):
    ones_v[pl.ds(j * 16, 16)] = jnp.full((16,), 1.0, jnp.float32)

  @pl.when(s == 0)
  def _():
    pltpu.sync_copy(zeros_hbm, acc_sh)
  pltpu.sync_copy(dst_hbm.at[wid], didx_v)
  plsc.subcore_barrier()

  G = 5

  def body(g, carry):
    j0 = g * G
    for j in range(G):
      pltpu.async_copy(ones_v, acc_sh.at[didx_v.at[j0 + j]], sem, add=True)
    for j in range(G):
      pltpu.make_async_copy(ones_v, acc_sh.at[didx_v.at[j0 + j]], sem).wait()
    return carry

  lax.fori_loop(0, NCHUNK // G, body, 0)
  plsc.subcore_barrier()

  @pl.when(s == 0)
  def _():
    pltpu.sync_copy(acc_sh, out_hbm.at[c])


@functools.partial(
    pl.kernel,
    out_type=jax.ShapeDtypeStruct((NC, N_PAD, D), jnp.float32),
    mesh=_mesh(),
    scratch_types=[
        pltpu.VMEM((NCHUNKA, CHA), jnp.int32),
        pltpu.VMEM((NCHUNKA, CHA), jnp.int32),
        pltpu.VMEM((CHA, D), jnp.float32),
        pltpu.VMEM_SHARED((N_PAD, D), jnp.float32),
        pltpu.SemaphoreType.DMA,
    ],
)
def _agg_kernel(src_hbm, dst_hbm, xp_hbm, zeros_hbm, out_hbm,
                sidx_v, didx_v, rows_v, acc_sh, sem0):
  c = lax.axis_index("c")
  s = lax.axis_index("s")
  wid = c * NS + s

  r0 = s * ROWS_PER_SUB
  pltpu.sync_copy(zeros_hbm.at[pl.ds(r0, ROWS_PER_SUB)],
                  acc_sh.at[pl.ds(r0, ROWS_PER_SUB)])
  pltpu.sync_copy(src_hbm.at[wid], sidx_v)
  pltpu.sync_copy(dst_hbm.at[wid], didx_v)
  plsc.subcore_barrier()

  def body(j, carry):
    pltpu.async_copy(xp_hbm.at[sidx_v.at[j]], rows_v, sem0).wait()
    pltpu.sync_copy(rows_v, acc_sh.at[didx_v.at[j]], add=True)
    return carry

  lax.fori_loop(0, NCHUNKA, body, 0)
  plsc.subcore_barrier()

  pltpu.sync_copy(acc_sh.at[pl.ds(r0, ROWS_PER_SUB)],
                  out_hbm.at[c, pl.ds(r0, ROWS_PER_SUB)])


def _scale_body(x_ref, dp_ref, o_ref):
  deg = 1.0 + dp_ref[:, 0:1] + dp_ref[:, 1:2]
  o_ref[...] = x_ref[...] * lax.rsqrt(deg)


def _scale_tc(x, dp):
  R = 2000
  grid = (N // R,)
  return pl.pallas_call(
      _scale_body,
      grid=grid,
      in_specs=[
          pl.BlockSpec((R, D), lambda i: (i, 0)),
          pl.BlockSpec((R, 2), lambda i: (i, 0)),
      ],
      out_specs=pl.BlockSpec((R, D), lambda i: (i, 0)),
      out_shape=jax.ShapeDtypeStruct((N, D), jnp.float32),
  )(x, dp)


def _finish_body(x_ref, p_ref, dp_ref, w_ref, b_ref, g_ref, be_ref, o_ref):
  deg = 1.0 + dp_ref[:, 0:1] + dp_ref[:, 1:2]
  dinv = lax.rsqrt(deg)
  x = x_ref[...]
  sagg = (p_ref[0] + p_ref[1] + x * dinv) * dinv
  y = jnp.dot(sagg, w_ref[...], preferred_element_type=jnp.float32)
  y = y + b_ref[...] + x
  mean = jnp.mean(y, axis=-1, keepdims=True)
  yc = y - mean
  var = jnp.mean(yc * yc, axis=-1, keepdims=True)
  yn = yc * lax.rsqrt(var + 1e-5) * g_ref[...] + be_ref[...]
  o_ref[...] = jnp.maximum(yn, 0.0)


def _finish_tc(x, accp, dp, W, b, gamma, beta):
  R = 2000
  grid = (N // R,)
  row = lambda i: (i, 0)
  full = lambda i: (0, 0)
  return pl.pallas_call(
      _finish_body,
      grid=grid,
      in_specs=[
          pl.BlockSpec((R, D), row),
          pl.BlockSpec((NC, R, D), lambda i: (0, i, 0)),
          pl.BlockSpec((R, 2), row),
          pl.BlockSpec((D, D), full),
          pl.BlockSpec((1, D), full),
          pl.BlockSpec((1, D), full),
          pl.BlockSpec((1, D), full),
      ],
      out_specs=pl.BlockSpec((R, D), row),
      out_shape=jax.ShapeDtypeStruct((N, D), jnp.float32),
  )(x, accp, dp, W, b, gamma, beta)


def kernel(x, edge_index, W, b, gamma, beta):
  ei = edge_index.astype(jnp.int32)
  dstd = ei[1].reshape(NW, NCHUNK, CH)
  pad = E_PAD - E
  srcp = jnp.concatenate([ei[0], jnp.zeros((pad,), jnp.int32)])
  dstp = jnp.concatenate([ei[1], jnp.full((pad,), N, jnp.int32)])
  srcp = srcp.reshape(NW, NCHUNKA, CHA)
  dstp = dstp.reshape(NW, NCHUNKA, CHA)
  zeros_nd = jnp.zeros((N_PAD, D), jnp.float32)
  zeros_n = jnp.zeros((N_PAD,), jnp.float32)

  degp = _deg_kernel(dstd, zeros_n)
  dp = degp[:, :N].T
  xp = _scale_tc(x, dp)
  accp = _agg_kernel(srcp, dstp, xp, zeros_nd)
  return _finish_tc(x, accp, dp, W,
                    b.reshape(1, D), gamma.reshape(1, D), beta.reshape(1, D))
